# baseline (speedup 1.0000x reference)
import functools

import jax
import jax.numpy as jnp
from jax import lax
from jax.experimental import pallas as pl
from jax.experimental.pallas import tpu as pltpu
from jax.experimental.pallas import tpu_sc as plsc

_N = 10000
_E = 320000
_D = 128
_NC = 2
_NS = 16
_EPT = _E // _NS
_CHG = 80
_CHZ = 40
_SEG = 2
_CPT = _EPT // _SEG // _CHG
_CPS = _EPT // _SEG // _CHZ
_HALF = 5120
_ACC = _HALF + 8
_RPTC = _HALF // _NS
_BR = 1000

_mesh = plsc.VectorSubcoreMesh(
    core_axis_name="c", subcore_axis_name="s", num_cores=_NC, num_subcores=_NS
)


@functools.partial(
    pl.kernel,
    out_type=jax.ShapeDtypeStruct((_NC, _HALF, _D), jnp.float32),
    mesh=_mesh,
    scratch_types=[
        pltpu.VMEM((_CPT, _CHG), jnp.int32),
        pltpu.VMEM((_CPS, _CHZ), jnp.int32),
        pltpu.VMEM((2, _CHG, _D), jnp.float32),
        pltpu.VMEM_SHARED((_ACC, _D), jnp.float32),
        pltpu.SemaphoreType.DMA,
    ],
)
def _agg_kernel(h_hbm, src_hbm, dstc_hbm, zeros_hbm, out_hbm,
                idx_s, idx_d, rows_v, acc_sc, gsem):
    c = lax.axis_index("c")
    s = lax.axis_index("s")
    r0 = s * _RPTC
    zer_v = rows_v.at[0, pl.ds(0, _CHZ)]
    pltpu.sync_copy(zeros_hbm, zer_v)
    for k in range(8):
        pltpu.sync_copy(zer_v, acc_sc.at[pl.ds(r0 + k * _CHZ, _CHZ)])

    @pl.when(s == 0)
    def _():
        pltpu.sync_copy(rows_v.at[0, pl.ds(0, 8)], acc_sc.at[pl.ds(_HALF, 8)])

    plsc.subcore_barrier()

    for seg in range(_SEG):
        pltpu.sync_copy(src_hbm.at[s, seg], idx_s)
        pltpu.sync_copy(dstc_hbm.at[c, s, seg], idx_d)

        pltpu.async_copy(h_hbm.at[idx_s.at[0]], rows_v.at[0], gsem)

        @pl.loop(0, _CPT)
        def _(j):
            b = j % 2

            @pl.when(j < _CPT - 1)
            def _():
                pltpu.async_copy(h_hbm.at[idx_s.at[j + 1]],
                                 rows_v.at[1 - b], gsem)

            pltpu.make_async_copy(h_hbm.at[idx_s.at[j]],
                                  rows_v.at[b], gsem).wait()
            pltpu.sync_copy(rows_v.at[b, pl.ds(0, _CHZ)],
                            acc_sc.at[idx_d.at[2 * j]], add=True)
            pltpu.sync_copy(rows_v.at[b, pl.ds(_CHZ, _CHZ)],
                            acc_sc.at[idx_d.at[2 * j + 1]], add=True)

    plsc.subcore_barrier()
    for k in range(8):
        pltpu.sync_copy(acc_sc.at[pl.ds(r0 + k * _CHZ, _CHZ)],
                        out_hbm.at[c, pl.ds(r0 + k * _CHZ, _CHZ)])


def _prep_body(s_ref, d_ref, sc_ref, dc_ref, hot_ref):
    for cc in range(_NC):
        lo = cc * _HALF
        for ref, out in ((s_ref, sc_ref), (d_ref, dc_ref)):
            v = ref[...]
            inh = jnp.logical_and(v >= lo, v < lo + _HALF)
            g = _HALF + jnp.bitwise_and(v, 7)
            out[cc] = jnp.where(inh, v - lo, g)
    hot_ref[...] = jnp.bitwise_and(s_ref[...], 7)


def _prep(src2, dst2):
    return pl.pallas_call(
        _prep_body,
        out_shape=[
            jax.ShapeDtypeStruct((_NC, _E // _D, _D), jnp.int32),
            jax.ShapeDtypeStruct((_NC, _E // _D, _D), jnp.int32),
            jax.ShapeDtypeStruct((_E // _D, _D), jnp.int32),
        ],
    )(src2, dst2)


def _mm1_body(x_ref, w_ref, dego_ref, degi_ref, h_ref, nsrc_ref, ndst_ref):
    deg_o = dego_ref[:, 0]
    deg_i = degi_ref[:, 0]
    nsrc = jnp.where(deg_o > 0, lax.rsqrt(deg_o), 0.0)
    ndst = jnp.where(deg_i > 0, lax.rsqrt(deg_i), 0.0)
    y = jnp.dot(x_ref[...], w_ref[...], preferred_element_type=jnp.float32)
    h_ref[...] = y * nsrc[:, None]
    nsrc_ref[...] = nsrc[:, None]
    ndst_ref[...] = ndst[:, None]


def _mm1(x, w, dego, degi):
    return pl.pallas_call(
        _mm1_body,
        grid=(_N // _BR,),
        in_specs=[
            pl.BlockSpec((_BR, _D), lambda i: (i, 0)),
            pl.BlockSpec((_D, _D), lambda i: (0, 0)),
            pl.BlockSpec((_BR, _D), lambda i: (i, 0)),
            pl.BlockSpec((_BR, _D), lambda i: (i, 0)),
        ],
        out_specs=[
            pl.BlockSpec((_BR, _D), lambda i: (i, 0)),
            pl.BlockSpec((_BR, 1), lambda i: (i, 0)),
            pl.BlockSpec((_BR, 1), lambda i: (i, 0)),
        ],
        out_shape=[
            jax.ShapeDtypeStruct((_N, _D), jnp.float32),
            jax.ShapeDtypeStruct((_N, 1), jnp.float32),
            jax.ShapeDtypeStruct((_N, 1), jnp.float32),
        ],
    )(x, w, dego, degi)


def _mid_body(agg_ref, ndst_ref, b_ref, nsrc_ref, w_ref, out_ref):
    x = agg_ref[...] * ndst_ref[...] + b_ref[...][None, :]
    x = jnp.maximum(x, 0.0)
    y = jnp.dot(x, w_ref[...], preferred_element_type=jnp.float32)
    out_ref[...] = y * nsrc_ref[...]


def _mid(agg, ndst, b, nsrc, w):
    return pl.pallas_call(
        _mid_body,
        grid=(_N // _BR,),
        in_specs=[
            pl.BlockSpec((_BR, _D), lambda i: (i, 0)),
            pl.BlockSpec((_BR, 1), lambda i: (i, 0)),
            pl.BlockSpec((_D,), lambda i: (0,)),
            pl.BlockSpec((_BR, 1), lambda i: (i, 0)),
            pl.BlockSpec((_D, _D), lambda i: (0, 0)),
        ],
        out_specs=pl.BlockSpec((_BR, _D), lambda i: (i, 0)),
        out_shape=jax.ShapeDtypeStruct((_N, _D), jnp.float32),
    )(agg, ndst, b, nsrc, w)


def _fin_body(agg_ref, ndst_ref, b_ref, out_ref):
    out_ref[...] = agg_ref[...] * ndst_ref[...] + b_ref[...][None, :]


def _fin(agg, ndst, b):
    return pl.pallas_call(
        _fin_body,
        grid=(_N // _BR,),
        in_specs=[
            pl.BlockSpec((_BR, _D), lambda i: (i, 0)),
            pl.BlockSpec((_BR, 1), lambda i: (i, 0)),
            pl.BlockSpec((_D,), lambda i: (0,)),
        ],
        out_specs=pl.BlockSpec((_BR, _D), lambda i: (i, 0)),
        out_shape=jax.ShapeDtypeStruct((_N, _D), jnp.float32),
    )(agg, ndst, b)


def kernel(inputs, edge_index, W1, b1, W2, b2, W3, b3):
    src = edge_index[0]
    dst = edge_index[1]
    srcc, dstc, hot = _prep(src.reshape(_E // _D, _D), dst.reshape(_E // _D, _D))
    srcc = srcc.reshape(_NC, _NS, _SEG, _CPS, _CHZ)
    dstc = dstc.reshape(_NC, _NS, _SEG, _CPS, _CHZ)
    src_g = src.reshape(_NS, _SEG, _CPT, _CHG)
    hot_g = hot.reshape(_NS, _SEG, _CPT, _CHG)
    zer = jnp.zeros((_CHZ, _D), jnp.float32)
    ones_h = jnp.ones((_N, _D), jnp.float32)

    dego = _agg_kernel(ones_h, hot_g, srcc, zer).reshape(_NC * _HALF, _D)
    degi = _agg_kernel(ones_h, hot_g, dstc, zer).reshape(_NC * _HALF, _D)
    h, nsrc, ndst = _mm1(inputs, W1, dego, degi)
    agg = _agg_kernel(h, src_g, dstc, zer).reshape(_NC * _HALF, _D)
    h = _mid(agg, ndst, b1, nsrc, W2)
    agg = _agg_kernel(h, src_g, dstc, zer).reshape(_NC * _HALF, _D)
    h = _mid(agg, ndst, b2, nsrc, W3)
    agg = _agg_kernel(h, src_g, dstc, zer).reshape(_NC * _HALF, _D)
    return _fin(agg, ndst, b3)


# speedup vs baseline: 7.4125x; 7.4125x over previous
import functools

import jax
import jax.numpy as jnp
from jax import lax
from jax.experimental import pallas as pl
from jax.experimental.pallas import tpu as pltpu
from jax.experimental.pallas import tpu_sc as plsc

_N = 10000
_E = 320000
_D = 128
_NC = 2
_NS = 16
_EPT = _E // _NS
_CHG = 80
_CHZ = 40
_SEG = 2
_CPT = _EPT // _SEG // _CHG
_CPS = _EPT // _SEG // _CHZ
_HALF = 5120
_ACC = _HALF + 8
_RPTC = _HALF // _NS
_BR = 1000

_mesh = plsc.VectorSubcoreMesh(
    core_axis_name="c", subcore_axis_name="s", num_cores=_NC, num_subcores=_NS
)


@functools.partial(
    pl.kernel,
    out_type=jax.ShapeDtypeStruct((_NC, _HALF, _D), jnp.float32),
    mesh=_mesh,
    scratch_types=[
        pltpu.VMEM((_CPT, _CHG), jnp.int32),
        pltpu.VMEM((_CPS, _CHZ), jnp.int32),
        pltpu.VMEM((3, _CHG, _D), jnp.float32),
        pltpu.VMEM_SHARED((_ACC, _D), jnp.float32),
        pltpu.SemaphoreType.DMA,
        pltpu.SemaphoreType.DMA,
    ],
)
def _agg_kernel(h_hbm, src_hbm, dstc_hbm, zeros_hbm, out_hbm,
                idx_s, idx_d, rows_v, acc_sc, gsem, ssem):
    c = lax.axis_index("c")
    s = lax.axis_index("s")
    r0 = s * _RPTC
    zer_v = rows_v.at[0, pl.ds(0, _CHZ)]
    pltpu.sync_copy(zeros_hbm, zer_v)
    for k in range(8):
        pltpu.sync_copy(zer_v, acc_sc.at[pl.ds(r0 + k * _CHZ, _CHZ)])

    @pl.when(s == 0)
    def _():
        pltpu.sync_copy(rows_v.at[0, pl.ds(0, 8)], acc_sc.at[pl.ds(_HALF, 8)])

    plsc.subcore_barrier()

    for seg in range(_SEG):
        pltpu.sync_copy(src_hbm.at[s, seg], idx_s)
        pltpu.sync_copy(dstc_hbm.at[c, s, seg], idx_d)

        pltpu.async_copy(h_hbm.at[idx_s.at[0]], rows_v.at[0], gsem)

        @pl.loop(0, _CPT)
        def _(j):
            b = j % 3

            @pl.when(j >= 2)
            def _():
                for _q in range(2):
                    pltpu.make_async_copy(rows_v.at[0, pl.ds(0, _CHZ)],
                                          acc_sc.at[idx_d.at[0]], ssem).wait()

            @pl.when(j < _CPT - 1)
            def _():
                pltpu.async_copy(h_hbm.at[idx_s.at[j + 1]],
                                 rows_v.at[(j + 1) % 3], gsem)

            pltpu.make_async_copy(h_hbm.at[idx_s.at[j]],
                                  rows_v.at[b], gsem).wait()
            pltpu.async_copy(rows_v.at[b, pl.ds(0, _CHZ)],
                             acc_sc.at[idx_d.at[2 * j]], ssem, add=True)
            pltpu.async_copy(rows_v.at[b, pl.ds(_CHZ, _CHZ)],
                             acc_sc.at[idx_d.at[2 * j + 1]], ssem, add=True)

        for _q in range(4):
            pltpu.make_async_copy(rows_v.at[0, pl.ds(0, _CHZ)],
                                  acc_sc.at[idx_d.at[0]], ssem).wait()

    plsc.subcore_barrier()
    for k in range(8):
        pltpu.sync_copy(acc_sc.at[pl.ds(r0 + k * _CHZ, _CHZ)],
                        out_hbm.at[c, pl.ds(r0 + k * _CHZ, _CHZ)])


def _prep_body(s_ref, d_ref, sc_ref, dc_ref):
    for cc in range(_NC):
        lo = cc * _HALF
        for ref, out in ((s_ref, sc_ref), (d_ref, dc_ref)):
            v = ref[...]
            inh = jnp.logical_and(v >= lo, v < lo + _HALF)
            g = _HALF + jnp.bitwise_and(v, 7)
            out[cc] = jnp.where(inh, v - lo, g)


def _prep(src2, dst2):
    return pl.pallas_call(
        _prep_body,
        out_shape=[
            jax.ShapeDtypeStruct((_NC, _E // _D, _D), jnp.int32),
            jax.ShapeDtypeStruct((_NC, _E // _D, _D), jnp.int32),
        ],
    )(src2, dst2)


def _mm1_body(x_ref, w_ref, dego_ref, degi_ref, h_ref, nsrc_ref, ndst_ref):
    deg_o = dego_ref[:, 0]
    deg_i = degi_ref[:, 0]
    nsrc = jnp.where(deg_o > 0, lax.rsqrt(deg_o), 0.0)
    ndst = jnp.where(deg_i > 0, lax.rsqrt(deg_i), 0.0)
    y = jnp.dot(x_ref[...], w_ref[...], preferred_element_type=jnp.float32)
    h_ref[...] = y * nsrc[:, None]
    nsrc_ref[...] = nsrc[:, None]
    ndst_ref[...] = ndst[:, None]


def _mm1(x, w, dego, degi):
    return pl.pallas_call(
        _mm1_body,
        grid=(_N // _BR,),
        in_specs=[
            pl.BlockSpec((_BR, _D), lambda i: (i, 0)),
            pl.BlockSpec((_D, _D), lambda i: (0, 0)),
            pl.BlockSpec((_BR, _D), lambda i: (i, 0)),
            pl.BlockSpec((_BR, _D), lambda i: (i, 0)),
        ],
        out_specs=[
            pl.BlockSpec((_BR, _D), lambda i: (i, 0)),
            pl.BlockSpec((_BR, 1), lambda i: (i, 0)),
            pl.BlockSpec((_BR, 1), lambda i: (i, 0)),
        ],
        out_shape=[
            jax.ShapeDtypeStruct((_N, _D), jnp.float32),
            jax.ShapeDtypeStruct((_N, 1), jnp.float32),
            jax.ShapeDtypeStruct((_N, 1), jnp.float32),
        ],
    )(x, w, dego, degi)


def _mid_body(agg_ref, ndst_ref, b_ref, nsrc_ref, w_ref, out_ref):
    x = agg_ref[...] * ndst_ref[...] + b_ref[...][None, :]
    x = jnp.maximum(x, 0.0)
    y = jnp.dot(x, w_ref[...], preferred_element_type=jnp.float32)
    out_ref[...] = y * nsrc_ref[...]


def _mid(agg, ndst, b, nsrc, w):
    return pl.pallas_call(
        _mid_body,
        grid=(_N // _BR,),
        in_specs=[
            pl.BlockSpec((_BR, _D), lambda i: (i, 0)),
            pl.BlockSpec((_BR, 1), lambda i: (i, 0)),
            pl.BlockSpec((_D,), lambda i: (0,)),
            pl.BlockSpec((_BR, 1), lambda i: (i, 0)),
            pl.BlockSpec((_D, _D), lambda i: (0, 0)),
        ],
        out_specs=pl.BlockSpec((_BR, _D), lambda i: (i, 0)),
        out_shape=jax.ShapeDtypeStruct((_N, _D), jnp.float32),
    )(agg, ndst, b, nsrc, w)


def _fin_body(agg_ref, ndst_ref, b_ref, out_ref):
    out_ref[...] = agg_ref[...] * ndst_ref[...] + b_ref[...][None, :]


def _fin(agg, ndst, b):
    return pl.pallas_call(
        _fin_body,
        grid=(_N // _BR,),
        in_specs=[
            pl.BlockSpec((_BR, _D), lambda i: (i, 0)),
            pl.BlockSpec((_BR, 1), lambda i: (i, 0)),
            pl.BlockSpec((_D,), lambda i: (0,)),
        ],
        out_specs=pl.BlockSpec((_BR, _D), lambda i: (i, 0)),
        out_shape=jax.ShapeDtypeStruct((_N, _D), jnp.float32),
    )(agg, ndst, b)


def kernel(inputs, edge_index, W1, b1, W2, b2, W3, b3):
    src = edge_index[0]
    dst = edge_index[1]
    srcc, dstc = _prep(src.reshape(_E // _D, _D), dst.reshape(_E // _D, _D))
    srcc = srcc.reshape(_NC, _NS, _SEG, _CPS, _CHZ)
    dstc = dstc.reshape(_NC, _NS, _SEG, _CPS, _CHZ)
    src_g = src.reshape(_NS, _SEG, _CPT, _CHG)
    zer = jnp.zeros((_CHZ, _D), jnp.float32)
    ones_h = jnp.ones((_N, _D), jnp.float32)

    dego = _agg_kernel(ones_h, src_g, srcc, zer).reshape(_NC * _HALF, _D)
    degi = _agg_kernel(ones_h, src_g, dstc, zer).reshape(_NC * _HALF, _D)
    h, nsrc, ndst = _mm1(inputs, W1, dego, degi)
    agg = _agg_kernel(h, src_g, dstc, zer).reshape(_NC * _HALF, _D)
    h = _mid(agg, ndst, b1, nsrc, W2)
    agg = _agg_kernel(h, src_g, dstc, zer).reshape(_NC * _HALF, _D)
    h = _mid(agg, ndst, b2, nsrc, W3)
    agg = _agg_kernel(h, src_g, dstc, zer).reshape(_NC * _HALF, _D)
    return _fin(agg, ndst, b3)
